# Initial kernel scaffold; baseline (speedup 1.0000x reference)
import jax
import jax.numpy as jnp
from jax.experimental import pallas as pl


def kernel(idx, emb_weight):
    raise NotImplementedError("write your pallas kernel here")


# speedup vs baseline: 1.6828x; 1.6828x over previous
import functools

import jax
import jax.numpy as jnp
from jax import lax
from jax.experimental import pallas as pl
from jax.experimental.pallas import tpu as pltpu
from jax.experimental.pallas import tpu_sc as plsc

VOCAB = 1000000
DIM = 64
B = 16384
L = 50

N = B * L
NC, NS = 2, 16
NW = NC * NS
CHUNK = 128
N_PER_W = N // NW
N_CHUNKS = N_PER_W // CHUNK


def _emb_body(idx_hbm, table_hbm, out_hbm, idx_v, rows_v, gsem):
    cid = lax.axis_index("c")
    sid = lax.axis_index("s")
    wid = sid * NC + cid

    pltpu.sync_copy(idx_hbm.at[wid], idx_v)

    def step(j, carry):
        cp = pltpu.async_copy(table_hbm.at[idx_v.at[j]], rows_v, gsem)
        cp.wait()
        pltpu.sync_copy(rows_v, out_hbm.at[wid, j])
        return carry

    lax.fori_loop(0, N_CHUNKS, step, 0)


@jax.jit
def _emb_lookup(idx_grouped, emb_weight):
    mesh = plsc.VectorSubcoreMesh(core_axis_name="c", subcore_axis_name="s")
    run = pl.kernel(
        _emb_body,
        out_type=jax.ShapeDtypeStruct((NW, N_CHUNKS, CHUNK, DIM), jnp.float32),
        mesh=mesh,
        scratch_types=[
            pltpu.VMEM((N_CHUNKS, CHUNK), jnp.int32),
            pltpu.VMEM((CHUNK, DIM), jnp.float32),
            pltpu.SemaphoreType.DMA,
        ],
        compiler_params=pltpu.CompilerParams(use_tc_tiling_on_sc=False),
    )
    return run(idx_grouped, emb_weight)


def kernel(idx, emb_weight):
    idx_grouped = idx.reshape(NW, N_CHUNKS, CHUNK).astype(jnp.int32)
    out = _emb_lookup(idx_grouped, emb_weight)
    return out.reshape(B, L, DIM)


# speedup vs baseline: 1.8748x; 1.1141x over previous
import jax
import jax.numpy as jnp
from jax import lax
from jax.experimental import pallas as pl
from jax.experimental.pallas import tpu as pltpu
from jax.experimental.pallas import tpu_sc as plsc

VOCAB = 1000000
DIM = 64
B = 16384
L = 50

N = B * L
NC, NS = 2, 16
NW = NC * NS
CHUNK = 128
N_PER_W = N // NW
N_CHUNKS = N_PER_W // CHUNK
K = 4
D = 2 * K


def _emb_body(idx_hbm, table_hbm, out_hbm, idx_v, rows_v, *sems):
    gsem = sems[:D]
    osem = sems[D:]
    cid = lax.axis_index("c")
    sid = lax.axis_index("s")
    wid = sid * NC + cid

    pltpu.sync_copy(idx_hbm.at[wid], idx_v)

    def gather_fire(j, b):
        pltpu.async_copy(table_hbm.at[idx_v.at[j]], rows_v.at[b], gsem[b])

    def gather_wait(b):
        pltpu.make_async_copy(
            table_hbm.at[pl.ds(0, CHUNK)], rows_v.at[b], gsem[b]
        ).wait()

    def out_fire(j, b):
        pltpu.async_copy(rows_v.at[b], out_hbm.at[wid, j], osem[b])

    def out_wait(j, b):
        pltpu.make_async_copy(rows_v.at[b], out_hbm.at[wid, j], osem[b]).wait()

    for b in range(K):
        gather_fire(b, b)
    for j in range(K):
        gather_fire(j + K, j + K)
        gather_wait(j % D)
        out_fire(j, j % D)

    n_main = N_CHUNKS - 2 * K

    def group(g, carry):
        j0 = K + g * D
        for i in range(D):
            b = (K + i) % D
            bf = (K + i + K) % D
            j = j0 + i
            out_wait(j - K, bf)
            gather_fire(j + K, bf)
            gather_wait(b)
            out_fire(j, b)
        return carry

    lax.fori_loop(0, n_main // D, group, 0)

    for i in range(K):
        j = N_CHUNKS - K + i
        b = j % D
        out_wait(j - K, (j + K) % D)
        gather_wait(b)
        out_fire(j, b)
    for i in range(K):
        j = N_CHUNKS - K + i
        out_wait(j, j % D)


@jax.jit
def _emb_lookup(idx_grouped, emb_weight):
    mesh = plsc.VectorSubcoreMesh(core_axis_name="c", subcore_axis_name="s")
    run = pl.kernel(
        _emb_body,
        out_type=jax.ShapeDtypeStruct((NW, N_CHUNKS, CHUNK, DIM), jnp.float32),
        mesh=mesh,
        scratch_types=[
            pltpu.VMEM((N_CHUNKS, CHUNK), jnp.int32),
            pltpu.VMEM((D, CHUNK, DIM), jnp.float32),
        ] + [pltpu.SemaphoreType.DMA] * (2 * D),
        compiler_params=pltpu.CompilerParams(use_tc_tiling_on_sc=False),
    )
    return run(idx_grouped, emb_weight)


def kernel(idx, emb_weight):
    idx_grouped = idx.reshape(NW, N_CHUNKS, CHUNK).astype(jnp.int32)
    out = _emb_lookup(idx_grouped, emb_weight)
    return out.reshape(B, L, DIM)
